# baseline (speedup 1.0000x reference)
import functools

import jax
import jax.numpy as jnp
from jax import lax
from jax.experimental import pallas as pl
from jax.experimental.pallas import tpu as pltpu
from jax.experimental.pallas import tpu_sc as plsc

N = 131328
B = 32
NC = 2
NS = 16
NW = NC * NS
L = 16
CH = N // NW
CH_PAD = ((CH + L - 1) // L) * L
NVEC = CH_PAD // L


def _sc_body(pred_hbm, target_hbm, mask_hbm, out_hbm,
             mask_v, w_v, p_v, t_v, acc_v):
    m_len = mask_v.shape[0]
    wid = lax.axis_index("s") * NC + lax.axis_index("c")
    c0 = wid * CH

    pltpu.sync_copy(mask_hbm, mask_v)

    zeros = jnp.zeros((L,), jnp.float32)

    def zero_body(i, carry):
        w_v[pl.ds(i * L, L)] = zeros
        return carry

    lax.fori_loop(0, NVEC, zero_body, 0)
    p_v[pl.ds(CH_PAD - L, L)] = zeros
    t_v[pl.ds(CH_PAD - L, L)] = zeros

    ones = jnp.ones((L,), jnp.float32)

    def scan_body(i, carry):
        mv = mask_v[pl.ds(i * L, L)]
        local = mv - c0
        valid = (local >= 0) & (local < CH)
        lc = jnp.minimum(jnp.maximum(local, 0), CH_PAD - 1)
        plsc.store_scatter(w_v, [lc], ones, mask=valid)
        return carry

    lax.fori_loop(0, m_len // L, scan_body, 0)

    def row_body(b, acc):
        base = b * N + c0
        pltpu.sync_copy(pred_hbm.at[pl.ds(base, CH)], p_v.at[pl.ds(0, CH)])
        pltpu.sync_copy(target_hbm.at[pl.ds(base, CH)], t_v.at[pl.ds(0, CH)])

        def vec_body(i, a):
            p = p_v[pl.ds(i * L, L)]
            t = t_v[pl.ds(i * L, L)]
            w = w_v[pl.ds(i * L, L)]
            return a + jnp.abs(p - t) * w

        return lax.fori_loop(0, NVEC, vec_body, acc)

    acc = lax.fori_loop(0, B, row_body, jnp.zeros((L,), jnp.float32))
    acc_v[...] = acc
    pltpu.sync_copy(acc_v, out_hbm.at[wid])


@functools.partial(jax.jit, static_argnames=("m_pad",))
def _run(pred_flat, target_flat, mask, m_pad):
    mesh = plsc.VectorSubcoreMesh(
        core_axis_name="c", subcore_axis_name="s", num_cores=NC,
        num_subcores=NS)
    k = pl.kernel(
        _sc_body,
        out_type=jax.ShapeDtypeStruct((NW, L), jnp.float32),
        mesh=mesh,
        compiler_params=pltpu.CompilerParams(needs_layout_passes=False),
        scratch_types=[
            pltpu.VMEM((m_pad,), jnp.int32),
            pltpu.VMEM((CH_PAD,), jnp.float32),
            pltpu.VMEM((CH_PAD,), jnp.float32),
            pltpu.VMEM((CH_PAD,), jnp.float32),
            pltpu.VMEM((L,), jnp.float32),
        ],
    )
    return k(pred_flat, target_flat, mask)


def kernel(pred, target, mask):
    m = mask.shape[0]
    m_pad = ((m + L - 1) // L) * L
    if m_pad != m:
        mask = jnp.pad(mask, (0, m_pad - m), constant_values=-1)
    partial = _run(pred.reshape(-1), target.reshape(-1), mask, m_pad)
    return jnp.sum(partial)


# speedup vs baseline: 2.1809x; 2.1809x over previous
import functools

import jax
import jax.numpy as jnp
from jax import lax
from jax.experimental import pallas as pl
from jax.experimental.pallas import tpu as pltpu
from jax.experimental.pallas import tpu_sc as plsc

N = 131328
B = 32
NC = 2
NS = 16
NW = NC * NS
L = 16
CH = N // NW
CH_PAD = ((CH + L - 1) // L) * L
NVEC = CH_PAD // L
NBUF = 4


def _sc_body(pred_hbm, target_hbm, mask_hbm, out_hbm,
             mask_v, lidx_v, acc_v, rest):
    m_len = mask_v.shape[0]
    nvm = m_len // L
    pbufs = rest[0:NBUF]
    tbufs = rest[NBUF:2 * NBUF]
    psems = rest[2 * NBUF:3 * NBUF]
    tsems = rest[3 * NBUF:4 * NBUF]

    wid = lax.axis_index("s") * NC + lax.axis_index("c")
    c0 = wid * CH

    def start_row(r, u):
        base = r * N + c0
        pltpu.async_copy(pred_hbm.at[pl.ds(base, CH)],
                         pbufs[u].at[pl.ds(0, CH)], psems[u])
        pltpu.async_copy(target_hbm.at[pl.ds(base, CH)],
                         tbufs[u].at[pl.ds(0, CH)], tsems[u])

    def wait_row(r, u):
        base = r * N + c0
        pltpu.make_async_copy(pred_hbm.at[pl.ds(base, CH)],
                              pbufs[u].at[pl.ds(0, CH)], psems[u]).wait()
        pltpu.make_async_copy(target_hbm.at[pl.ds(base, CH)],
                              tbufs[u].at[pl.ds(0, CH)], tsems[u]).wait()

    zeros = jnp.zeros((L,), jnp.float32)
    for u in range(NBUF):
        pbufs[u][pl.ds(CH_PAD - L, L)] = zeros
        tbufs[u][pl.ds(CH_PAD - L, L)] = zeros

    for u in range(NBUF - 1):
        start_row(u, u)

    pltpu.sync_copy(mask_hbm, mask_v)

    n_steps = max(1, (nvm + 1).bit_length())

    def lower_bound_vec(x):
        def step(_, carry):
            pos, rem = carry
            half = rem // 2
            mid = pos + half
            probe = jnp.min(mask_v[pl.ds(jnp.minimum(mid, nvm - 1) * L, L)])
            active = rem > 0
            go_right = active & (probe < x)
            pos = jnp.where(go_right, mid + 1, pos)
            rem = jnp.where(active,
                            jnp.where(go_right, rem - half - 1, half),
                            0)
            return pos, rem

        pos, _ = lax.fori_loop(0, n_steps, step,
                               (jnp.int32(0), jnp.int32(nvm)))
        return pos

    jlo = jnp.maximum(lower_bound_vec(c0) - 1, 0)
    jhi = lower_bound_vec(c0 + CH)
    nv = jhi - jlo

    pad_idx = jnp.full((L,), CH_PAD - 1, jnp.int32)

    def build_body(k, carry):
        mv = mask_v[pl.ds((jlo + k) * L, L)]
        local = mv - c0
        valid = (local >= 0) & (local < CH)
        lidx_v[pl.ds(k * L, L)] = jnp.where(valid, local, CH_PAD - 1)
        return carry

    lax.fori_loop(0, nv, build_body, 0)
    for e in range(NBUF - 1):
        lidx_v[pl.ds((nv + e) * L, L)] = pad_idx

    nv4 = (nv + 3) // 4

    def make_row_compute(u):
        def vec4_body(k, a):
            for q in range(4):
                idx = lidx_v[pl.ds((k * 4 + q) * L, L)]
                p = plsc.load_gather(pbufs[u], [idx])
                t = plsc.load_gather(tbufs[u], [idx])
                a = a + jnp.abs(p - t)
            return a

        return vec4_body

    def ring_body(j, acc):
        for u in range(NBUF):
            r = j * NBUF + u
            start_row(jnp.minimum(r + NBUF - 1, B - 1), (u + NBUF - 1) % NBUF)
            wait_row(r, u)
            acc = lax.fori_loop(0, nv4, make_row_compute(u), acc)
        return acc

    acc = lax.fori_loop(0, B // NBUF, ring_body, jnp.zeros((L,), jnp.float32))

    for u in range(NBUF - 1):
        wait_row(B - 1, u)

    acc_v[...] = acc
    pltpu.sync_copy(acc_v, out_hbm.at[wid])


def _body_wrapper(pred_hbm, target_hbm, mask_hbm, out_hbm,
                  mask_v, lidx_v, acc_v, *rest):
    _sc_body(pred_hbm, target_hbm, mask_hbm, out_hbm,
             mask_v, lidx_v, acc_v, rest)


@functools.partial(jax.jit, static_argnames=("m_pad",))
def _run(pred_flat, target_flat, mask, m_pad):
    mesh = plsc.VectorSubcoreMesh(
        core_axis_name="c", subcore_axis_name="s", num_cores=NC,
        num_subcores=NS)
    k = pl.kernel(
        _body_wrapper,
        out_type=jax.ShapeDtypeStruct((NW, L), jnp.float32),
        mesh=mesh,
        compiler_params=pltpu.CompilerParams(needs_layout_passes=False),
        scratch_types=[
            pltpu.VMEM((m_pad,), jnp.int32),
            pltpu.VMEM((m_pad + NBUF * L,), jnp.int32),
            pltpu.VMEM((L,), jnp.float32),
        ] + [pltpu.VMEM((CH_PAD,), jnp.float32) for _ in range(2 * NBUF)]
          + [pltpu.SemaphoreType.DMA for _ in range(2 * NBUF)],
    )
    return k(pred_flat, target_flat, mask)


def kernel(pred, target, mask):
    m = mask.shape[0]
    m_pad = ((m + L - 1) // L) * L
    if m_pad != m:
        mask = jnp.pad(mask, (0, m_pad - m), constant_values=N)
    partial = _run(pred.reshape(-1), target.reshape(-1), mask, m_pad)
    return jnp.sum(partial)


# speedup vs baseline: 3.1005x; 1.4216x over previous
import functools

import jax
import jax.numpy as jnp
from jax import lax
from jax.experimental import pallas as pl
from jax.experimental.pallas import tpu as pltpu
from jax.experimental.pallas import tpu_sc as plsc

N = 131328
B = 32
NC = 2
NS = 16
NW = NC * NS
L = 16
CH = N // NW
TILES = 34
SPAN = TILES * 128
HALF = SPAN // 2
NTC = N // 128
NG = B // 8
CAPV = HALF // L + 2 + 3
SENT = jnp.int32(HALF)


def _sc_body(pred_hbm, target_hbm, mask_hbm, out_hbm,
             mask_v, lidx_a, lidx_b, acc_v,
             p0, p1, t0, t1, psem0, psem1, tsem0, tsem1):
    m_len = mask_v.shape[0]
    nvm = m_len // L

    wid = lax.axis_index("s") * NC + lax.axis_index("c")
    c0 = wid * CH
    tc_s = jnp.minimum((c0 // 128), NTC - TILES)
    cs = tc_s * 128

    pbufs, tbufs = (p0, p1), (t0, t1)
    psems, tsems = (psem0, psem1), (tsem0, tsem1)

    def unit_src(i):
        h, g = divmod(i, NG)
        return (pl.ds(8 * g, 8), pl.ds(cs + h * HALF, HALF))

    def start_unit(i):
        rs, cs_ = unit_src(i)
        pltpu.async_copy(pred_hbm.at[rs, cs_], pbufs[i % 2], psems[i % 2])
        pltpu.async_copy(target_hbm.at[rs, cs_], tbufs[i % 2], tsems[i % 2])

    def wait_unit(i):
        rs, cs_ = unit_src(i)
        pltpu.make_async_copy(pred_hbm.at[rs, cs_], pbufs[i % 2],
                              psems[i % 2]).wait()
        pltpu.make_async_copy(target_hbm.at[rs, cs_], tbufs[i % 2],
                              tsems[i % 2]).wait()

    start_unit(0)

    pltpu.sync_copy(mask_hbm, mask_v)

    n_steps = max(1, (nvm + 1).bit_length())

    def lower_bound_vec(x):
        def step(_, carry):
            pos, rem = carry
            half = rem // 2
            mid = pos + half
            probe = jnp.min(mask_v[pl.ds(jnp.minimum(mid, nvm - 1) * L, L)])
            active = rem > 0
            go_right = active & (probe < x)
            pos = jnp.where(go_right, mid + 1, pos)
            rem = jnp.where(active,
                            jnp.where(go_right, rem - half - 1, half),
                            0)
            return pos, rem

        pos, _ = lax.fori_loop(0, n_steps, step,
                               (jnp.int32(0), jnp.int32(nvm)))
        return pos

    ja = jnp.maximum(lower_bound_vec(c0) - 1, 0)
    jmid = lower_bound_vec(cs + HALF)
    jb = jnp.maximum(jmid - 1, 0)
    jhi = lower_bound_vec(c0 + CH)
    nva = jmid - ja
    nvb = jhi - jb

    def make_build(jlo, lidx, lo_col, hi_col, base):
        def body(k, carry):
            mv = mask_v[pl.ds((jlo + k) * L, L)]
            valid = (mv >= lo_col) & (mv < hi_col)
            lidx[pl.ds(k * L, L)] = jnp.where(valid, mv - base, SENT)
            return carry

        return body

    lax.fori_loop(0, nva, make_build(ja, lidx_a, c0, cs + HALF, cs), 0)
    lax.fori_loop(0, nvb, make_build(jb, lidx_b, cs + HALF, c0 + CH,
                                     cs + HALF), 0)
    pad = jnp.full((L,), SENT, jnp.int32)
    for e in range(3):
        lidx_a[pl.ds((nva + e) * L, L)] = pad
        lidx_b[pl.ds((nvb + e) * L, L)] = pad

    n4a = (nva + 3) // 4
    n4b = (nvb + 3) // 4
    zero = jnp.zeros((L,), jnp.float32)

    acc = zero
    for i in range(2 * NG):
        if i + 1 < 2 * NG:
            start_unit(i + 1)
        wait_unit(i)
        u = i % 2
        lidx = lidx_a if i < NG else lidx_b
        n4 = n4a if i < NG else n4b

        for rr in range(8):
            row_v = jnp.full((L,), rr, jnp.int32)

            def vec4_body(k, a, _lidx=lidx, _u=u, _row=row_v):
                for q in range(4):
                    cv = _lidx[pl.ds((k * 4 + q) * L, L)]
                    ok = cv < HALF
                    cc = jnp.minimum(cv, HALF - 1)
                    p = plsc.load_gather(pbufs[_u], [_row, cc])
                    t = plsc.load_gather(tbufs[_u], [_row, cc])
                    a = a + jnp.where(ok, jnp.abs(p - t), 0.0)
                return a

            acc = lax.fori_loop(0, n4, vec4_body, acc)

    acc_v[...] = acc
    pltpu.sync_copy(acc_v, out_hbm.at[wid])


@functools.partial(jax.jit, static_argnames=("m_pad",))
def _run(pred2d, target2d, mask, m_pad):
    mesh = plsc.VectorSubcoreMesh(
        core_axis_name="c", subcore_axis_name="s", num_cores=NC,
        num_subcores=NS)
    k = pl.kernel(
        _sc_body,
        out_type=jax.ShapeDtypeStruct((NW, L), jnp.float32),
        mesh=mesh,
        compiler_params=pltpu.CompilerParams(needs_layout_passes=False),
        scratch_types=[
            pltpu.VMEM((m_pad,), jnp.int32),
            pltpu.VMEM((CAPV * L,), jnp.int32),
            pltpu.VMEM((CAPV * L,), jnp.int32),
            pltpu.VMEM((L,), jnp.float32),
        ] + [pltpu.VMEM((8, HALF), jnp.float32) for _ in range(4)]
          + [pltpu.SemaphoreType.DMA for _ in range(4)],
    )
    return k(pred2d, target2d, mask)


def kernel(pred, target, mask):
    m = mask.shape[0]
    m_pad = ((m + L - 1) // L) * L
    if m_pad != m:
        mask = jnp.pad(mask, (0, m_pad - m), constant_values=N)
    partial = _run(pred, target, mask, m_pad)
    return jnp.sum(partial)


# speedup vs baseline: 3.7345x; 1.2045x over previous
import functools

import jax
import jax.numpy as jnp
from jax import lax
from jax.experimental import pallas as pl
from jax.experimental.pallas import tpu as pltpu
from jax.experimental.pallas import tpu_sc as plsc

N = 131328
B = 32
NC = 2
NS = 16
NW = NC * NS
L = 16
CH = N // NW
TILES = 33
SPAN = TILES * 128
THIRD = SPAN // 3
NTC = N // 128
NG = B // 8
NU = 3 * NG
NBUF = 3
CAPV = THIRD // L + 2 + 3
SENT = jnp.int32(THIRD)


def _sc_body(pred_hbm, target_hbm, mask_hbm, out_hbm,
             mask_v, lidx0, lidx1, lidx2, acc_v,
             p0, p1, p2, t0, t1, t2,
             psem0, psem1, psem2, tsem0, tsem1, tsem2):
    m_len = mask_v.shape[0]
    nvm = m_len // L

    wid = lax.axis_index("s") * NC + lax.axis_index("c")
    c0 = wid * CH
    tc_s = jnp.minimum((c0 // 128), NTC - TILES)
    cs = tc_s * 128

    pbufs, tbufs = (p0, p1, p2), (t0, t1, t2)
    psems, tsems = (psem0, psem1, psem2), (tsem0, tsem1, tsem2)
    lidxs = (lidx0, lidx1, lidx2)

    def unit_src(i):
        t, g = divmod(i, NG)
        return (pl.ds(8 * g, 8), pl.ds(cs + t * THIRD, THIRD))

    def start_unit(i):
        rs, cs_ = unit_src(i)
        u = i % NBUF
        pltpu.async_copy(pred_hbm.at[rs, cs_], pbufs[u], psems[u])
        pltpu.async_copy(target_hbm.at[rs, cs_], tbufs[u], tsems[u])

    def wait_unit(i):
        rs, cs_ = unit_src(i)
        u = i % NBUF
        pltpu.make_async_copy(pred_hbm.at[rs, cs_], pbufs[u], psems[u]).wait()
        pltpu.make_async_copy(target_hbm.at[rs, cs_], tbufs[u], tsems[u]).wait()

    start_unit(0)
    start_unit(1)

    pltpu.sync_copy(mask_hbm, mask_v)

    n_steps = max(1, (nvm + 1).bit_length())

    def lower_bound_vec(x):
        def step(_, carry):
            pos, rem = carry
            half = rem // 2
            mid = pos + half
            probe = jnp.min(mask_v[pl.ds(jnp.minimum(mid, nvm - 1) * L, L)])
            active = rem > 0
            go_right = active & (probe < x)
            pos = jnp.where(go_right, mid + 1, pos)
            rem = jnp.where(active,
                            jnp.where(go_right, rem - half - 1, half),
                            0)
            return pos, rem

        pos, _ = lax.fori_loop(0, n_steps, step,
                               (jnp.int32(0), jnp.int32(nvm)))
        return pos

    bounds = [jnp.maximum(c0, cs), cs + THIRD, cs + 2 * THIRD, c0 + CH]
    jpos = [lower_bound_vec(b) for b in bounds]
    jlos = [jnp.maximum(jpos[t] - 1, 0) for t in range(3)]
    nvs = [jpos[t + 1] - jlos[t] for t in range(3)]

    pad = jnp.full((L,), SENT, jnp.int32)
    for t in range(3):
        lo_col = bounds[t] if t == 0 else jnp.maximum(bounds[t], c0)
        hi_col = jnp.minimum(bounds[t + 1], c0 + CH)
        base = cs + t * THIRD
        lidx = lidxs[t]

        def build_body(k, carry, _lidx=lidx, _jlo=jlos[t], _lo=lo_col,
                       _hi=hi_col, _base=base):
            mv = mask_v[pl.ds((_jlo + k) * L, L)]
            valid = (mv >= _lo) & (mv < _hi)
            _lidx[pl.ds(k * L, L)] = jnp.where(valid, mv - _base, SENT)
            return carry

        lax.fori_loop(0, nvs[t], build_body, 0)
        for e in range(3):
            lidx[pl.ds((nvs[t] + e) * L, L)] = pad

    n4s = [(nvs[t] + 3) // 4 for t in range(3)]
    zero = jnp.zeros((L,), jnp.float32)

    acc = zero
    for i in range(NU):
        if i + NBUF - 1 < NU:
            start_unit(i + NBUF - 1)
        wait_unit(i)
        u = i % NBUF
        t = i // NG

        def row_body(rr, a, _lidx=lidxs[t], _n4=n4s[t], _u=u):
            row_v = jnp.full((L,), 0, jnp.int32) + rr

            def vec4_body(k, aa):
                for q in range(4):
                    cv = _lidx[pl.ds((k * 4 + q) * L, L)]
                    ok = cv < THIRD
                    cc = jnp.minimum(cv, THIRD - 1)
                    p = plsc.load_gather(pbufs[_u], [row_v, cc])
                    tt = plsc.load_gather(tbufs[_u], [row_v, cc])
                    aa = aa + jnp.where(ok, jnp.abs(p - tt), 0.0)
                return aa

            return lax.fori_loop(0, _n4, vec4_body, a)

        acc = lax.fori_loop(0, 8, row_body, acc)

    acc_v[...] = acc
    pltpu.sync_copy(acc_v, out_hbm.at[wid])


@functools.partial(jax.jit, static_argnames=("m_pad",))
def _run(pred2d, target2d, mask, m_pad):
    mesh = plsc.VectorSubcoreMesh(
        core_axis_name="c", subcore_axis_name="s", num_cores=NC,
        num_subcores=NS)
    k = pl.kernel(
        _sc_body,
        out_type=jax.ShapeDtypeStruct((NW, L), jnp.float32),
        mesh=mesh,
        compiler_params=pltpu.CompilerParams(needs_layout_passes=False),
        scratch_types=[
            pltpu.VMEM((m_pad,), jnp.int32),
            pltpu.VMEM((CAPV * L,), jnp.int32),
            pltpu.VMEM((CAPV * L,), jnp.int32),
            pltpu.VMEM((CAPV * L,), jnp.int32),
            pltpu.VMEM((L,), jnp.float32),
        ] + [pltpu.VMEM((8, THIRD), jnp.float32) for _ in range(6)]
          + [pltpu.SemaphoreType.DMA for _ in range(6)],
    )
    return k(pred2d, target2d, mask)


def kernel(pred, target, mask):
    m = mask.shape[0]
    m_pad = ((m + L - 1) // L) * L
    if m_pad != m:
        mask = jnp.pad(mask, (0, m_pad - m), constant_values=N)
    partial = _run(pred, target, mask, m_pad)
    return jnp.sum(partial)


# speedup vs baseline: 3.8689x; 1.0360x over previous
import functools

import jax
import jax.numpy as jnp
from jax import lax
from jax.experimental import pallas as pl
from jax.experimental.pallas import tpu as pltpu
from jax.experimental.pallas import tpu_sc as plsc

N = 131328
B = 32
NC = 2
NS = 16
NW = NC * NS
L = 16
CH = N // NW
TILES = 33
SPAN = TILES * 128
THIRD = SPAN // 3
NTC = N // 128
NG = B // 8
NU = 3 * NG
NBUF = 4
CAPW = THIRD // L + 2
CAPV = CAPW + 3
SENT = jnp.int32(THIRD)


def _sc_body(pred_hbm, target_hbm, mask_hbm, l0_hbm, out_hbm,
             l0_v, win_v, lidx0, lidx1, lidx2, acc_v,
             p0, p1, p2, p3, t0, t1, t2, t3,
             psem0, psem1, psem2, psem3, tsem0, tsem1, tsem2, tsem3):
    m_len = mask_hbm.shape[0]
    nvm = m_len // L

    wid = lax.axis_index("s") * NC + lax.axis_index("c")
    c0 = wid * CH
    tc_s = jnp.minimum((c0 // 128), NTC - TILES)
    cs = tc_s * 128

    pbufs, tbufs = (p0, p1, p2, p3), (t0, t1, t2, t3)
    psems = (psem0, psem1, psem2, psem3)
    tsems = (tsem0, tsem1, tsem2, tsem3)
    lidxs = (lidx0, lidx1, lidx2)

    def unit_src(i):
        t, g = divmod(i, NG)
        return (pl.ds(8 * g, 8), pl.ds(cs + t * THIRD, THIRD))

    def start_unit(i):
        rs, cs_ = unit_src(i)
        u = i % NBUF
        pltpu.async_copy(pred_hbm.at[rs, cs_], pbufs[u], psems[u])
        pltpu.async_copy(target_hbm.at[rs, cs_], tbufs[u], tsems[u])

    def wait_unit(i):
        rs, cs_ = unit_src(i)
        u = i % NBUF
        pltpu.make_async_copy(pred_hbm.at[rs, cs_], pbufs[u], psems[u]).wait()
        pltpu.make_async_copy(target_hbm.at[rs, cs_], tbufs[u], tsems[u]).wait()

    for i in range(NBUF - 1):
        start_unit(i)

    pltpu.sync_copy(l0_hbm, l0_v)

    n_steps = max(1, (nvm + 1).bit_length())

    def lower_bound_vec(x):
        def step(_, carry):
            pos, rem = carry
            half = rem // 2
            mid = pos + half
            probe = jnp.min(l0_v[pl.ds(jnp.minimum(mid, nvm - 1), L)])
            active = rem > 0
            go_right = active & (probe < x)
            pos = jnp.where(go_right, mid + 1, pos)
            rem = jnp.where(active,
                            jnp.where(go_right, rem - half - 1, half),
                            0)
            return pos, rem

        pos, _ = lax.fori_loop(0, n_steps, step,
                               (jnp.int32(0), jnp.int32(nvm)))
        return pos

    bounds = [c0, cs + THIRD, cs + 2 * THIRD, c0 + CH]
    jpos = [lower_bound_vec(b) for b in bounds]
    jlos = [jnp.maximum(jpos[t] - 1, 0) for t in range(3)]
    nvs = [jpos[t + 1] - jlos[t] for t in range(3)]

    pad = jnp.full((L,), SENT, jnp.int32)
    for t in range(3):
        lo_col = bounds[t] if t == 0 else jnp.maximum(bounds[t], c0)
        hi_col = jnp.minimum(bounds[t + 1], c0 + CH)
        base = cs + t * THIRD
        lidx = lidxs[t]
        wstart = jnp.minimum(jlos[t], nvm - CAPW)
        pltpu.sync_copy(mask_hbm.at[pl.ds(wstart * L, CAPW * L)], win_v)
        rel = jlos[t] - wstart

        def build_body(k, carry, _lidx=lidx, _rel=rel, _lo=lo_col,
                       _hi=hi_col, _base=base):
            mv = win_v[pl.ds((_rel + k) * L, L)]
            valid = (mv >= _lo) & (mv < _hi)
            _lidx[pl.ds(k * L, L)] = jnp.where(valid, mv - _base, SENT)
            return carry

        lax.fori_loop(0, nvs[t], build_body, 0)
        for e in range(3):
            lidx[pl.ds((nvs[t] + e) * L, L)] = pad

    n4s = [(nvs[t] + 3) // 4 for t in range(3)]
    zero = jnp.zeros((L,), jnp.float32)

    acc = zero
    for i in range(NU):
        if i + NBUF - 1 < NU:
            start_unit(i + NBUF - 1)
        wait_unit(i)
        u = i % NBUF
        t = i // NG

        def row_body(rr, a, _lidx=lidxs[t], _n4=n4s[t], _u=u):
            row_v = jnp.full((L,), 0, jnp.int32) + rr

            def vec4_body(k, aa):
                for q in range(4):
                    cv = _lidx[pl.ds((k * 4 + q) * L, L)]
                    ok = cv < THIRD
                    cc = jnp.minimum(cv, THIRD - 1)
                    p = plsc.load_gather(pbufs[_u], [row_v, cc])
                    tt = plsc.load_gather(tbufs[_u], [row_v, cc])
                    aa = aa + jnp.where(ok, jnp.abs(p - tt), 0.0)
                return aa

            return lax.fori_loop(0, _n4, vec4_body, a)

        acc = lax.fori_loop(0, 8, row_body, acc)

    acc_v[...] = acc
    pltpu.sync_copy(acc_v, out_hbm.at[wid])


@functools.partial(jax.jit, static_argnames=("m_pad",))
def _run(pred2d, target2d, mask, l0, m_pad):
    mesh = plsc.VectorSubcoreMesh(
        core_axis_name="c", subcore_axis_name="s", num_cores=NC,
        num_subcores=NS)
    k = pl.kernel(
        _sc_body,
        out_type=jax.ShapeDtypeStruct((NW, L), jnp.float32),
        mesh=mesh,
        compiler_params=pltpu.CompilerParams(needs_layout_passes=False),
        scratch_types=[
            pltpu.VMEM((m_pad // L + L,), jnp.int32),
            pltpu.VMEM((CAPW * L,), jnp.int32),
            pltpu.VMEM((CAPV * L,), jnp.int32),
            pltpu.VMEM((CAPV * L,), jnp.int32),
            pltpu.VMEM((CAPV * L,), jnp.int32),
            pltpu.VMEM((L,), jnp.float32),
        ] + [pltpu.VMEM((8, THIRD), jnp.float32) for _ in range(2 * NBUF)]
          + [pltpu.SemaphoreType.DMA for _ in range(2 * NBUF)],
    )
    return k(pred2d, target2d, mask, l0)


def kernel(pred, target, mask):
    m = mask.shape[0]
    m_pad = ((m + L - 1) // L) * L
    m_pad = max(m_pad, CAPW * L)
    if m_pad != m:
        mask = jnp.pad(mask, (0, m_pad - m), constant_values=N)
    l0 = jnp.pad(mask[::L], (0, L), constant_values=N)
    partial = _run(pred, target, mask, l0, m_pad)
    return jnp.sum(partial)
